# baseline (speedup 1.0000x reference)
import functools

import jax
import jax.numpy as jnp
from jax import lax
from jax.experimental import pallas as pl
from jax.experimental.pallas import tpu as pltpu
from jax.experimental.pallas import tpu_sc as plsc

DIM = 64
B_ROWS = 16384
B_COLS = 50

_info = plsc.get_sparse_core_info()
NC = _info.num_cores
NS = _info.num_subcores
NW = NC * NS
D0_PER_W = B_ROWS // NW

IB = 128
K = D0_PER_W // IB
NCHUNK = B_COLS
NBUF = 2
NGROUP = NCHUNK // NBUF


@functools.partial(
    pl.kernel,
    mesh=plsc.VectorSubcoreMesh(core_axis_name="c", subcore_axis_name="s"),
    out_type=jax.ShapeDtypeStruct((B_ROWS, B_COLS, DIM), jnp.float32),
    scratch_types=[
        pltpu.VMEM((NBUF, D0_PER_W), jnp.int32),
        pltpu.VMEM((NBUF, D0_PER_W, DIM), jnp.float32),
        pltpu.SemaphoreType.DMA,
        pltpu.SemaphoreType.DMA,
    ],
    compiler_params=pltpu.CompilerParams(use_tc_tiling_on_sc=False),
)
def _gather_kernel(idxT_hbm, table_hbm, out_hbm, idx_v, rows_v, gat_sem, out_sem):
    wid = lax.axis_index("s") * NC + lax.axis_index("c")
    d0 = wid * D0_PER_W

    def start_gather(s, buf):
        pltpu.sync_copy(idxT_hbm.at[s, pl.ds(d0, D0_PER_W)], idx_v.at[buf])
        for j in range(K):
            pltpu.async_copy(
                table_hbm.at[idx_v.at[buf, pl.ds(j * IB, IB)]],
                rows_v.at[buf, pl.ds(j * IB, IB)],
                gat_sem,
            )

    def wait_gather(s, buf):
        for j in range(K):
            pltpu.make_async_copy(
                table_hbm.at[idx_v.at[buf, pl.ds(j * IB, IB)]],
                rows_v.at[buf, pl.ds(j * IB, IB)],
                gat_sem,
            ).wait()

    def wait_writeback(buf):
        pltpu.make_async_copy(
            rows_v.at[buf], out_hbm.at[pl.ds(d0, D0_PER_W), 0, :], out_sem
        ).wait()

    start_gather(0, 0)

    def group(g, _):
        s0 = g * NBUF
        for b in range(NBUF):
            s = s0 + b
            nb = (b + 1) % NBUF

            @pl.when(s + 1 < NCHUNK)
            def _():
                @pl.when(s + 1 >= NBUF)
                def _():
                    wait_writeback(nb)
                start_gather(s + 1, nb)

            wait_gather(s, b)
            pltpu.async_copy(
                rows_v.at[b], out_hbm.at[pl.ds(d0, D0_PER_W), s, :], out_sem
            )
        return _

    lax.fori_loop(0, NGROUP, group, None, unroll=False)

    for b in range(NBUF):
        wait_writeback(b)


def kernel(input_, weight):
    idxT = input_.T.astype(jnp.int32)
    return _gather_kernel(idxT, weight)
